# baseline (speedup 1.0000x reference)
import functools
import math

import jax
import jax.numpy as jnp
from jax import lax
from jax.experimental import pallas as pl
from jax.experimental.pallas import tpu as pltpu
from jax.experimental.pallas import tpu_sc as plsc

_NC = 2
_NS = 16
_NW = _NC * _NS
_ROWS = 100
_MW = 120


def _sc_call(x2, mpad, pe_ext, n_chunks):
    cpw = n_chunks // _NW
    scale = math.sqrt(128.0)
    mesh = plsc.VectorSubcoreMesh(
        core_axis_name="c", subcore_axis_name="s", num_cores=_NC, num_subcores=_NS
    )

    @functools.partial(
        pl.kernel,
        out_type=jax.ShapeDtypeStruct((n_chunks, _ROWS, 128), jnp.float32),
        mesh=mesh,
        scratch_types=[
            pltpu.VMEM((201, 128), jnp.float32),
            pltpu.VMEM((2, _ROWS, 128), jnp.float32),
            pltpu.VMEM((2, _ROWS, 128), jnp.float32),
            pltpu.VMEM((2, _MW), jnp.float32),
            pltpu.SemaphoreType.DMA,
            pltpu.SemaphoreType.DMA,
            pltpu.SemaphoreType.DMA,
            pltpu.SemaphoreType.DMA,
        ],
    )
    def run(x_hbm, m_hbm, pe_hbm, out_hbm, pe_v, bin_v, bout_v, m_v,
            sin0, sin1, sout0, sout1):
        c = lax.axis_index("c")
        s = lax.axis_index("s")
        wid = s * _NC + c
        base = wid * cpw
        pltpu.sync_copy(pe_hbm, pe_v)
        pad = [pe_v[200, pl.ds(16 * j, 16)] for j in range(8)]
        sins = (sin0, sin1)
        souts = (sout0, sout1)

        def start_in(r, p):
            pltpu.async_copy(x_hbm.at[r], bin_v.at[p], sins[p])
            pltpu.async_copy(m_hbm.at[r], m_v.at[p], sins[p])

        def wait_in(r, p):
            pltpu.make_async_copy(x_hbm.at[r], bin_v.at[p], sins[p]).wait()
            pltpu.make_async_copy(m_hbm.at[r], m_v.at[p], sins[p]).wait()

        def start_out(r, p):
            pltpu.async_copy(bout_v.at[p], out_hbm.at[r], souts[p])

        def wait_out(r, p):
            pltpu.make_async_copy(bout_v.at[p], out_hbm.at[r], souts[p]).wait()

        start_in(base + 0, 0)
        start_in(base + 1, 1)

        def compute(p):
            loff = _ROWS * p

            def lbody(l, carry):
                mf = m_v[p, pl.ds(l, 16)][0]
                s1 = 1.0 - mf
                for j in range(8):
                    sl = pl.ds(16 * j, 16)
                    xv = bin_v[p, l, sl]
                    pv = pe_v[loff + l, sl]
                    bout_v[p, l, sl] = xv * scale + pv * s1 + pad[j] * mf
                return carry

            lax.fori_loop(0, _ROWS, lbody, 0)

        def outer(gg, carry):
            for p in (0, 1):
                g = 2 * gg + p
                r = base + g

                wait_in(r, p)

                @pl.when(gg > 0)
                def _():
                    wait_out(r - 2, p)

                compute(p)
                start_out(r, p)

                @pl.when(g + 2 < cpw)
                def _():
                    start_in(r + 2, p)
            return carry

        lax.fori_loop(0, cpw // 2, outer, 0)
        wait_out(base + cpw - 2, 0)
        wait_out(base + cpw - 1, 1)

    return run(x2, mpad, pe_ext)


def kernel(x, mask, pe):
    B, L, D = x.shape
    n_chunks = B * L // _ROWS
    pad_row = jax.lax.slice_in_dim(pe, pe.shape[0] - 1, pe.shape[0], axis=0)
    pe_ext = jnp.concatenate([jax.lax.slice_in_dim(pe, 0, L, axis=0), pad_row], 0)
    x2 = x.reshape(n_chunks, _ROWS, D)
    mask_f = mask.astype(x.dtype).reshape(n_chunks, _ROWS)
    mpad = jnp.pad(mask_f, ((0, 0), (0, _MW - _ROWS)))
    out2 = _sc_call(x2, mpad, pe_ext, n_chunks)
    return out2.reshape(B, L, D)


# speedup vs baseline: 1.8491x; 1.8491x over previous
import functools
import math

import jax
import jax.numpy as jnp
from jax import lax
from jax.experimental import pallas as pl
from jax.experimental.pallas import tpu as pltpu
from jax.experimental.pallas import tpu_sc as plsc

_NC = 2
_NS = 16
_NW = _NC * _NS
_ROWS = 100
_MW = 120

_SPLIT = 3392
_BB = 64


def _tc_body(x_ref, m_ref, pe_ref, diff_ref, o_ref):
    x = x_ref[...]
    m = m_ref[...]
    pe = pe_ref[...]
    diff = diff_ref[...]
    scale = math.sqrt(x.shape[-1])
    m3 = jax.lax.broadcast_in_dim(m, x.shape, (0, 1))
    o_ref[...] = x * scale + (pe[None, :, :] - m3 * diff[None, :, :])


def _tc_call(x_tc, mask_tc, pe_l, diff, B, L, D):
    grid = (_SPLIT // _BB,)
    return pl.pallas_call(
        _tc_body,
        grid=grid,
        in_specs=[
            pl.BlockSpec((_BB, L, D), lambda i: (i, 0, 0)),
            pl.BlockSpec((_BB, L), lambda i: (i, 0)),
            pl.BlockSpec((L, D), lambda i: (0, 0)),
            pl.BlockSpec((L, D), lambda i: (0, 0)),
        ],
        out_specs=pl.BlockSpec((_BB, L, D), lambda i: (i, 0, 0)),
        out_shape=jax.ShapeDtypeStruct((B, L, D), x_tc.dtype),
    )(x_tc, mask_tc, pe_l, diff)


def _sc_call(x2, mpad, pe_ext, n_chunks):
    cpw = n_chunks // _NW
    scale = math.sqrt(128.0)
    mesh = plsc.VectorSubcoreMesh(
        core_axis_name="c", subcore_axis_name="s", num_cores=_NC, num_subcores=_NS
    )

    @functools.partial(
        pl.kernel,
        out_type=jax.ShapeDtypeStruct((n_chunks, _ROWS, 128), jnp.float32),
        mesh=mesh,
        scratch_types=[
            pltpu.VMEM((201, 128), jnp.float32),
            pltpu.VMEM((2, _ROWS, 128), jnp.float32),
            pltpu.VMEM((2, _ROWS, 128), jnp.float32),
            pltpu.VMEM((2, _MW), jnp.float32),
            pltpu.SemaphoreType.DMA,
            pltpu.SemaphoreType.DMA,
            pltpu.SemaphoreType.DMA,
            pltpu.SemaphoreType.DMA,
        ],
    )
    def run(x_hbm, m_hbm, pe_hbm, out_hbm, pe_v, bin_v, bout_v, m_v,
            sin0, sin1, sout0, sout1):
        c = lax.axis_index("c")
        s = lax.axis_index("s")
        wid = s * _NC + c
        base = wid * cpw
        pltpu.sync_copy(pe_hbm, pe_v)
        pad = [pe_v[200, pl.ds(16 * j, 16)] for j in range(8)]
        sins = (sin0, sin1)
        souts = (sout0, sout1)

        def start_in(r, p):
            pltpu.async_copy(x_hbm.at[r], bin_v.at[p], sins[p])
            pltpu.async_copy(m_hbm.at[r], m_v.at[p], sins[p])

        def wait_in(r, p):
            pltpu.make_async_copy(x_hbm.at[r], bin_v.at[p], sins[p]).wait()
            pltpu.make_async_copy(m_hbm.at[r], m_v.at[p], sins[p]).wait()

        def start_out(r, p):
            pltpu.async_copy(bout_v.at[p], out_hbm.at[r], souts[p])

        def wait_out(r, p):
            pltpu.make_async_copy(bout_v.at[p], out_hbm.at[r], souts[p]).wait()

        start_in(base + 0, 0)
        start_in(base + 1, 1)

        def compute(p):
            loff = _ROWS * p

            def lbody(l, carry):
                mf = m_v[p, pl.ds(l, 16)][0]
                s1 = 1.0 - mf
                for j in range(8):
                    sl = pl.ds(16 * j, 16)
                    xv = bin_v[p, l, sl]
                    pv = pe_v[loff + l, sl]
                    bout_v[p, l, sl] = xv * scale + pv * s1 + pad[j] * mf
                return carry

            lax.fori_loop(0, _ROWS, lbody, 0)

        def outer(gg, carry):
            for p in (0, 1):
                g = 2 * gg + p
                r = base + g

                wait_in(r, p)

                @pl.when(gg > 0)
                def _():
                    wait_out(r - 2, p)

                compute(p)
                start_out(r, p)

                @pl.when(g + 2 < cpw)
                def _():
                    start_in(r + 2, p)
            return carry

        lax.fori_loop(0, cpw // 2, outer, 0)
        wait_out(base + cpw - 2, 0)
        wait_out(base + cpw - 1, 1)

    return run(x2, mpad, pe_ext)


def kernel(x, mask, pe):
    B, L, D = x.shape
    pad_row = jax.lax.slice_in_dim(pe, pe.shape[0] - 1, pe.shape[0], axis=0)
    pe_l = jax.lax.slice_in_dim(pe, 0, L, axis=0)
    diff = pe_l - pad_row
    mask_f = mask.astype(x.dtype)

    b_sc = B - _SPLIT
    n_chunks = b_sc * L // _ROWS
    pe_ext = jnp.concatenate([pe_l, pad_row], 0)
    x_sc = x[_SPLIT:].reshape(n_chunks, _ROWS, D)
    m_sc = mask_f[_SPLIT:].reshape(n_chunks, _ROWS)
    mpad = jnp.pad(m_sc, ((0, 0), (0, _MW - _ROWS)))
    out_sc = _sc_call(x_sc, mpad, pe_ext, n_chunks)

    out_full = _tc_call(x[:_SPLIT], mask_f[:_SPLIT], pe_l, diff, B, L, D)

    return lax.dynamic_update_slice(
        out_full, out_sc.reshape(b_sc, L, D), (_SPLIT, 0, 0)
    )


# speedup vs baseline: 5.0251x; 2.7176x over previous
import math

import jax
import jax.numpy as jnp
from jax.experimental import pallas as pl


def _body(x_ref, m_ref, pe_ref, diff_ref, o_ref):
    x = x_ref[...]
    m = m_ref[...]
    pe = pe_ref[...]
    diff = diff_ref[...]
    scale = math.sqrt(x.shape[-1])
    m3 = jax.lax.broadcast_in_dim(m, x.shape, (0, 1))
    o_ref[...] = x * scale + (pe[None, :, :] - m3 * diff[None, :, :])


def kernel(x, mask, pe):
    B, L, D = x.shape
    pad_row = jax.lax.slice_in_dim(pe, pe.shape[0] - 1, pe.shape[0], axis=0)
    pe_l = jax.lax.slice_in_dim(pe, 0, L, axis=0)
    diff = pe_l - pad_row
    mask_f = mask.astype(x.dtype)
    bB = 128
    grid = (B // bB,)
    return pl.pallas_call(
        _body,
        grid=grid,
        in_specs=[
            pl.BlockSpec((bB, L, D), lambda i: (i, 0, 0)),
            pl.BlockSpec((bB, L), lambda i: (i, 0)),
            pl.BlockSpec((L, D), lambda i: (0, 0)),
            pl.BlockSpec((L, D), lambda i: (0, 0)),
        ],
        out_specs=pl.BlockSpec((bB, L, D), lambda i: (i, 0, 0)),
        out_shape=jax.ShapeDtypeStruct((B, L, D), x.dtype),
    )(x, mask_f, pe_l, diff)


# speedup vs baseline: 5.0434x; 1.0036x over previous
import math

import jax
import jax.numpy as jnp
from jax.experimental import pallas as pl


def _body(x_ref, m_ref, pe_ref, diff_ref, o_ref):
    x = x_ref[...]
    m = m_ref[...].astype(x_ref.dtype)
    pe = pe_ref[...]
    diff = diff_ref[...]
    scale = math.sqrt(x.shape[-1])
    m3 = jax.lax.broadcast_in_dim(m, x.shape, (0, 1))
    o_ref[...] = x * scale + (pe[None, :, :] - m3 * diff[None, :, :])


def kernel(x, mask, pe):
    B, L, D = x.shape
    pad_row = jax.lax.slice_in_dim(pe, pe.shape[0] - 1, pe.shape[0], axis=0)
    pe_l = jax.lax.slice_in_dim(pe, 0, L, axis=0)
    diff = pe_l - pad_row
    bB = 128
    grid = (B // bB,)
    return pl.pallas_call(
        _body,
        grid=grid,
        in_specs=[
            pl.BlockSpec((bB, L, D), lambda i: (i, 0, 0)),
            pl.BlockSpec((bB, L), lambda i: (i, 0)),
            pl.BlockSpec((L, D), lambda i: (0, 0)),
            pl.BlockSpec((L, D), lambda i: (0, 0)),
        ],
        out_specs=pl.BlockSpec((bB, L, D), lambda i: (i, 0, 0)),
        out_shape=jax.ShapeDtypeStruct((B, L, D), x.dtype),
    )(x, mask, pe_l, diff)
